# Initial kernel scaffold; baseline (speedup 1.0000x reference)
import jax
import jax.numpy as jnp
from jax.experimental import pallas as pl


def kernel(x, edges, W1, b1, W2, b2):
    raise NotImplementedError("write your pallas kernel here")


# speedup vs baseline: 47.7946x; 47.7946x over previous
import functools

import jax
import jax.numpy as jnp
from jax import lax
from jax.experimental import pallas as pl
from jax.experimental.pallas import tpu as pltpu
from jax.experimental.pallas import tpu_sc as plsc

NC, NS = 2, 16
NW = NC * NS
L = 16

T, N, D, OUT = 3, 100000, 4, 2
E = 3200000
C8 = 2 * T + 2

ER = E // 128
ER_BASE = ER // NW
ER_REM = ER % NW
MB = 16
NMAC = ER_BASE // MB

NODE_STRIDE = 3120
NODE_CHUNK = N - (NW - 1) * NODE_STRIDE
NGROUPS = NODE_CHUNK // L
ACC_ROWS = N // NS

_mesh = plsc.VectorSubcoreMesh(
    core_axis_name="c", subcore_axis_name="s", num_cores=NC, num_subcores=NS
)
_params = pltpu.CompilerParams(
    needs_layout_passes=False, use_tc_tiling_on_sc=False
)


def _wid():
    return lax.axis_index("c") * NS + lax.axis_index("s")


def _proj_body(x_hbm, wflat_hbm, z_hbm, d_hbm, xbuf, zbuf, dbuf, wbuf):
    wid = _wid()
    n0 = wid * NODE_STRIDE
    pltpu.sync_copy(wflat_hbm, wbuf)
    wv = [wbuf[pl.ds(k * L, L)] for k in range(4)]

    def _sc(i):
        return wv[i // L][i % L]

    iota = lax.iota(jnp.int32, L)
    for t in range(T):
        pltpu.sync_copy(x_hbm.at[t, pl.ds(n0, NODE_CHUNK), :], xbuf)
        w1s = [[_sc(t * 8 + dd * 2 + o) for o in range(OUT)] for dd in range(D)]
        w2s = [[_sc(24 + t * 8 + dd * 2 + o) for o in range(OUT)] for dd in range(D)]
        bs = [_sc(48 + t * 2 + o) + _sc(54 + t * 2 + o) for o in range(OUT)]

        def body(g, carry):
            rows = g * L + iota
            xs = [plsc.load_gather(xbuf, [rows, jnp.full((L,), dd, jnp.int32)])
                  for dd in range(D)]
            for o in range(OUT):
                zv = xs[0] * w2s[0][o]
                dv = xs[0] * w1s[0][o]
                for dd in range(1, D):
                    zv = zv + xs[dd] * w2s[dd][o]
                    dv = dv + xs[dd] * w1s[dd][o]
                dv = dv + bs[o]
                ch = jnp.full((L,), 2 * t + o, jnp.int32)
                plsc.store_scatter(zbuf, [rows, ch], zv)
                plsc.store_scatter(dbuf, [rows, ch], dv)
            if t == 0:
                zz = jnp.zeros((L,), jnp.float32)
                for ch in (2 * T, 2 * T + 1):
                    chv = jnp.full((L,), ch, jnp.int32)
                    plsc.store_scatter(zbuf, [rows, chv], zz)
                    plsc.store_scatter(dbuf, [rows, chv], zz)
            return carry

        lax.fori_loop(0, NGROUPS, body, 0)
    pltpu.sync_copy(zbuf, z_hbm.at[pl.ds(n0, NODE_CHUNK), :])
    pltpu.sync_copy(dbuf, d_hbm.at[pl.ds(n0, NODE_CHUNK), :])


_proj = functools.partial(
    pl.kernel,
    out_type=(
        jax.ShapeDtypeStruct((N, C8), jnp.float32),
        jax.ShapeDtypeStruct((N, C8), jnp.float32),
    ),
    mesh=_mesh,
    compiler_params=_params,
    scratch_types=[
        pltpu.VMEM((NODE_CHUNK, D), jnp.float32),
        pltpu.VMEM((NODE_CHUNK, C8), jnp.float32),
        pltpu.VMEM((NODE_CHUNK, C8), jnp.float32),
        pltpu.VMEM((4 * L,), jnp.float32),
    ],
)(_proj_body)


def _scat_body(z_hbm, edges_hbm, zero_hbm, parts_hbm,
               sidx, didx, rows, acc, gsem0, gsem1, ssem):
    c = lax.axis_index("c")
    s = lax.axis_index("s")
    wid = c * NS + s
    pltpu.sync_copy(zero_hbm, acc.at[pl.ds(s * ACC_ROWS, ACC_ROWS), :])
    plsc.subcore_barrier()

    r0 = wid * ER_BASE + jnp.minimum(wid, ER_REM)
    cnt = ER_BASE + jnp.where(wid < ER_REM, 1, 0)
    gsems = (gsem0, gsem1)

    def _load_idx(p, r):
        pltpu.sync_copy(edges_hbm.at[0, pl.ds(r * 128, MB * 128)], sidx.at[p])
        pltpu.sync_copy(edges_hbm.at[1, pl.ds(r * 128, MB * 128)], didx.at[p])

    def _si(p, j):
        return sidx.at[p, pl.ds(j * 128, 128)]

    def _di(p, j):
        return didx.at[p, pl.ds(j * 128, 128)]

    def _fire_gathers(p):
        for j in range(MB):
            pltpu.async_copy(z_hbm.at[_si(p, j)], rows.at[p, j], gsems[p])

    def _wait_gathers(p):
        for j in range(MB):
            pltpu.make_async_copy(
                z_hbm.at[_si(p, j)], rows.at[p, j], gsems[p]
            ).wait()

    def _scatter(p):
        cps = [
            pltpu.async_copy(rows.at[p, j], acc.at[_di(p, j)], ssem, add=True)
            for j in range(MB)
        ]
        for cp in cps:
            cp.wait()

    _load_idx(0, r0)
    _fire_gathers(0)
    _load_idx(1, r0 + MB)
    _fire_gathers(1)

    def mbody(m, carry):
        for b in range(2):
            k = 2 * m + b
            r = r0 + k * MB
            _wait_gathers(b)
            _scatter(b)
            nxt = r + 2 * MB

            @pl.when(k + 2 < NMAC)
            def _():
                _load_idx(b, nxt)
                _fire_gathers(b)

        return carry

    lax.fori_loop(0, NMAC // 2, mbody, 0)

    def tbody(r, carry):
        pltpu.sync_copy(edges_hbm.at[0, pl.ds(r * 128, 128)], _si(0, 0))
        pltpu.sync_copy(edges_hbm.at[1, pl.ds(r * 128, 128)], _di(0, 0))
        pltpu.async_copy(z_hbm.at[_si(0, 0)], rows.at[0, 0], gsem0).wait()
        pltpu.sync_copy(rows.at[0, 0], acc.at[_di(0, 0)], add=True)
        return carry

    lax.fori_loop(r0 + NMAC * MB, r0 + cnt, tbody, 0)
    plsc.subcore_barrier()
    pltpu.sync_copy(acc.at[pl.ds(s * ACC_ROWS, ACC_ROWS), :],
                    parts_hbm.at[c, pl.ds(s * ACC_ROWS, ACC_ROWS), :])


_scat = functools.partial(
    pl.kernel,
    out_type=jax.ShapeDtypeStruct((NC, N, C8), jnp.float32),
    mesh=_mesh,
    compiler_params=_params,
    scratch_types=[
        pltpu.VMEM((2, MB * 128), jnp.int32),
        pltpu.VMEM((2, MB * 128), jnp.int32),
        pltpu.VMEM((2, MB, 128, C8), jnp.float32),
        pltpu.VMEM_SHARED((N, C8), jnp.float32),
        pltpu.SemaphoreType.DMA,
        pltpu.SemaphoreType.DMA,
        pltpu.SemaphoreType.DMA,
    ],
)(_scat_body)


def _epi_body(d_hbm, parts_hbm, out_hbm, dbuf, p0, p1):
    wid = _wid()
    n0 = wid * NODE_STRIDE
    pltpu.sync_copy(d_hbm.at[pl.ds(n0, NODE_CHUNK), :], dbuf)
    pltpu.sync_copy(parts_hbm.at[0, pl.ds(n0, NODE_CHUNK), :], p0)
    pltpu.sync_copy(parts_hbm.at[1, pl.ds(n0, NODE_CHUNK), :], p1)

    iota = lax.iota(jnp.int32, L)

    def gbody(g, carry):
        rows = g * L + iota
        for t in range(T):
            for o in range(OUT):
                ch = jnp.full((L,), 2 * t + o, jnp.int32)
                h = (plsc.load_gather(dbuf, [rows, ch])
                     + plsc.load_gather(p0, [rows, ch])
                     + plsc.load_gather(p1, [rows, ch]))
                h = jnp.maximum(h, 0.0)
                sg = 1.0 / (1.0 + jnp.exp(-h))
                plsc.store_scatter(dbuf, [rows, ch], sg)
        return carry

    lax.fori_loop(0, NGROUPS, gbody, 0)
    for t in range(T):
        pltpu.sync_copy(dbuf.at[:, pl.ds(2 * t, OUT)],
                        out_hbm.at[t, pl.ds(n0, NODE_CHUNK), :])


_epi = functools.partial(
    pl.kernel,
    out_type=jax.ShapeDtypeStruct((T, N, OUT), jnp.float32),
    mesh=_mesh,
    compiler_params=_params,
    scratch_types=[
        pltpu.VMEM((NODE_CHUNK, C8), jnp.float32),
        pltpu.VMEM((NODE_CHUNK, C8), jnp.float32),
        pltpu.VMEM((NODE_CHUNK, C8), jnp.float32),
    ],
)(_epi_body)


def kernel(x, edges, W1, b1, W2, b2):
    zeros = jnp.zeros((ACC_ROWS, C8), jnp.float32)
    wflat = jnp.concatenate([
        W1.reshape(-1), W2.reshape(-1), b1.reshape(-1), b2.reshape(-1),
        jnp.zeros((4,), jnp.float32),
    ])
    z, d = _proj(x, wflat)
    parts = _scat(z, edges, zeros)
    return _epi(d, parts)


# speedup vs baseline: 79.2136x; 1.6574x over previous
import functools

import jax
import jax.numpy as jnp
from jax import lax
from jax.experimental import pallas as pl
from jax.experimental.pallas import tpu as pltpu
from jax.experimental.pallas import tpu_sc as plsc

NC, NS = 2, 16
NW = NC * NS
L = 16

T, N, D, OUT = 3, 100000, 4, 2
E = 3200000
C8 = 2 * T + 2

ER = E // 128
ER_BASE = ER // NW
ER_REM = ER % NW
MB = 16
NMAC = ER_BASE // MB

NODE_STRIDE = 3120
NODE_CHUNK = N - (NW - 1) * NODE_STRIDE
NGROUPS = NODE_CHUNK // L
ACC_ROWS = N // NS

_mesh = plsc.VectorSubcoreMesh(
    core_axis_name="c", subcore_axis_name="s", num_cores=NC, num_subcores=NS
)
_params = pltpu.CompilerParams(
    needs_layout_passes=False, use_tc_tiling_on_sc=False
)


def _wid():
    return lax.axis_index("c") * NS + lax.axis_index("s")


def _proj_body(x_hbm, wflat_hbm, z_hbm, d_hbm, xbuf, zbuf, dbuf, wbuf):
    wid = _wid()
    n0 = wid * NODE_STRIDE
    pltpu.sync_copy(wflat_hbm, wbuf)
    wv = [wbuf[pl.ds(k * L, L)] for k in range(4)]

    def _sc(i):
        return wv[i // L][i % L]

    iota = lax.iota(jnp.int32, L)
    for t in range(T):
        pltpu.sync_copy(x_hbm.at[t, pl.ds(n0, NODE_CHUNK), :], xbuf)
        w1s = [[_sc(t * 8 + dd * 2 + o) for o in range(OUT)] for dd in range(D)]
        w2s = [[_sc(24 + t * 8 + dd * 2 + o) for o in range(OUT)] for dd in range(D)]
        bs = [_sc(48 + t * 2 + o) + _sc(54 + t * 2 + o) for o in range(OUT)]

        def body(g, carry):
            rows = g * L + iota
            xs = [plsc.load_gather(xbuf, [rows, jnp.full((L,), dd, jnp.int32)])
                  for dd in range(D)]
            for o in range(OUT):
                zv = xs[0] * w2s[0][o]
                dv = xs[0] * w1s[0][o]
                for dd in range(1, D):
                    zv = zv + xs[dd] * w2s[dd][o]
                    dv = dv + xs[dd] * w1s[dd][o]
                dv = dv + bs[o]
                ch = jnp.full((L,), 2 * t + o, jnp.int32)
                plsc.store_scatter(zbuf, [rows, ch], zv)
                plsc.store_scatter(dbuf, [rows, ch], dv)
            if t == 0:
                zz = jnp.zeros((L,), jnp.float32)
                for ch in (2 * T, 2 * T + 1):
                    chv = jnp.full((L,), ch, jnp.int32)
                    plsc.store_scatter(zbuf, [rows, chv], zz)
                    plsc.store_scatter(dbuf, [rows, chv], zz)
            return carry

        lax.fori_loop(0, NGROUPS, body, 0)
    pltpu.sync_copy(zbuf, z_hbm.at[pl.ds(n0, NODE_CHUNK), :])
    pltpu.sync_copy(dbuf, d_hbm.at[pl.ds(n0, NODE_CHUNK), :])


_proj = functools.partial(
    pl.kernel,
    out_type=(
        jax.ShapeDtypeStruct((N, C8), jnp.float32),
        jax.ShapeDtypeStruct((N, C8), jnp.float32),
    ),
    mesh=_mesh,
    compiler_params=_params,
    scratch_types=[
        pltpu.VMEM((NODE_CHUNK, D), jnp.float32),
        pltpu.VMEM((NODE_CHUNK, C8), jnp.float32),
        pltpu.VMEM((NODE_CHUNK, C8), jnp.float32),
        pltpu.VMEM((4 * L,), jnp.float32),
    ],
)(_proj_body)


def _scat_body(z_hbm, esrc_hbm, edst_hbm, zero_hbm, parts_hbm,
               sidx, didx, rows, acc, gsem0, gsem1, ssem):
    c = lax.axis_index("c")
    s = lax.axis_index("s")
    wid = c * NS + s
    pltpu.sync_copy(zero_hbm, acc.at[pl.ds(s * ACC_ROWS, ACC_ROWS), :])
    plsc.subcore_barrier()

    r0 = wid * ER_BASE + jnp.minimum(wid, ER_REM)
    cnt = ER_BASE + jnp.where(wid < ER_REM, 1, 0)
    gsems = (gsem0, gsem1)

    def _load_idx(p, r):
        pltpu.sync_copy(esrc_hbm.at[pl.ds(r * 128, MB * 128)], sidx.at[p])
        pltpu.sync_copy(edst_hbm.at[pl.ds(r * 128, MB * 128)], didx.at[p])

    def _si(p, j):
        return sidx.at[p, pl.ds(j * 128, 128)]

    def _di(p, j):
        return didx.at[p, pl.ds(j * 128, 128)]

    def _fire_gathers(p):
        for j in range(MB):
            pltpu.async_copy(z_hbm.at[_si(p, j)], rows.at[p, j], gsems[p])

    def _wait_gathers(p):
        for j in range(MB):
            pltpu.make_async_copy(
                z_hbm.at[_si(p, j)], rows.at[p, j], gsems[p]
            ).wait()

    def _scatter(p):
        cps = [
            pltpu.async_copy(rows.at[p, j], acc.at[_di(p, j)], ssem, add=True)
            for j in range(MB)
        ]
        for cp in cps:
            cp.wait()

    _load_idx(0, r0)
    _fire_gathers(0)
    _load_idx(1, r0 + MB)
    _fire_gathers(1)

    def mbody(m, carry):
        for b in range(2):
            k = 2 * m + b
            r = r0 + k * MB
            _wait_gathers(b)
            _scatter(b)
            nxt = r + 2 * MB

            @pl.when(k + 2 < NMAC)
            def _():
                _load_idx(b, nxt)
                _fire_gathers(b)

        return carry

    lax.fori_loop(0, NMAC // 2, mbody, 0)

    def tbody(r, carry):
        pltpu.sync_copy(esrc_hbm.at[pl.ds(r * 128, 128)], _si(0, 0))
        pltpu.sync_copy(edst_hbm.at[pl.ds(r * 128, 128)], _di(0, 0))
        pltpu.async_copy(z_hbm.at[_si(0, 0)], rows.at[0, 0], gsem0).wait()
        pltpu.sync_copy(rows.at[0, 0], acc.at[_di(0, 0)], add=True)
        return carry

    lax.fori_loop(r0 + NMAC * MB, r0 + cnt, tbody, 0)
    plsc.subcore_barrier()
    pltpu.sync_copy(acc.at[pl.ds(s * ACC_ROWS, ACC_ROWS), :],
                    parts_hbm.at[c, pl.ds(s * ACC_ROWS, ACC_ROWS), :])


_scat = functools.partial(
    pl.kernel,
    out_type=jax.ShapeDtypeStruct((NC, N, C8), jnp.float32),
    mesh=_mesh,
    compiler_params=_params,
    scratch_types=[
        pltpu.VMEM((2, MB * 128), jnp.int32),
        pltpu.VMEM((2, MB * 128), jnp.int32),
        pltpu.VMEM((2, MB, 128, C8), jnp.float32),
        pltpu.VMEM_SHARED((N, C8), jnp.float32),
        pltpu.SemaphoreType.DMA,
        pltpu.SemaphoreType.DMA,
        pltpu.SemaphoreType.DMA,
    ],
)(_scat_body)


def _epi_body(d_hbm, parts_hbm, out_hbm, dbuf, p0, p1, ob0, ob1, ob2):
    wid = _wid()
    n0 = wid * NODE_STRIDE
    pltpu.sync_copy(d_hbm.at[pl.ds(n0, NODE_CHUNK), :], dbuf)
    pltpu.sync_copy(parts_hbm.at[0, pl.ds(n0, NODE_CHUNK), :], p0)
    pltpu.sync_copy(parts_hbm.at[1, pl.ds(n0, NODE_CHUNK), :], p1)

    iota = lax.iota(jnp.int32, L)

    obufs = (ob0, ob1, ob2)

    def gbody(g, carry):
        rows = g * L + iota
        for t in range(T):
            for o in range(OUT):
                ch = jnp.full((L,), 2 * t + o, jnp.int32)
                h = (plsc.load_gather(dbuf, [rows, ch])
                     + plsc.load_gather(p0, [rows, ch])
                     + plsc.load_gather(p1, [rows, ch]))
                h = jnp.maximum(h, 0.0)
                sg = 1.0 / (1.0 + jnp.exp(-h))
                plsc.store_scatter(obufs[t], [rows * OUT + o], sg)
        return carry

    lax.fori_loop(0, NGROUPS, gbody, 0)
    for t in range(T):
        pltpu.sync_copy(obufs[t],
                        out_hbm.at[t, pl.ds(n0 * OUT, NODE_CHUNK * OUT)])


_epi = functools.partial(
    pl.kernel,
    out_type=jax.ShapeDtypeStruct((T, N * OUT), jnp.float32),
    mesh=_mesh,
    compiler_params=_params,
    scratch_types=[
        pltpu.VMEM((NODE_CHUNK, C8), jnp.float32),
        pltpu.VMEM((NODE_CHUNK, C8), jnp.float32),
        pltpu.VMEM((NODE_CHUNK, C8), jnp.float32),
        pltpu.VMEM((NODE_CHUNK * OUT,), jnp.float32),
        pltpu.VMEM((NODE_CHUNK * OUT,), jnp.float32),
        pltpu.VMEM((NODE_CHUNK * OUT,), jnp.float32),
    ],
)(_epi_body)


def kernel(x, edges, W1, b1, W2, b2):
    zeros = jnp.zeros((ACC_ROWS, C8), jnp.float32)
    wflat = jnp.concatenate([
        W1.reshape(-1), W2.reshape(-1), b1.reshape(-1), b2.reshape(-1),
        jnp.zeros((4,), jnp.float32),
    ])
    z, d = _proj(x, wflat)
    parts = _scat(z, edges[0], edges[1], zeros)
    return _epi(d, parts).reshape(T, N, OUT)
